# baseline (speedup 1.0000x reference)
import functools

import jax
import jax.numpy as jnp
from jax import lax
from jax.experimental import pallas as pl
from jax.experimental.pallas import tpu as pltpu
from jax.experimental.pallas import tpu_sc as plsc

N_CLS = 1000
N_CTX = 16
D = 768
SEQ = 77
SUF = SEQ - 1 - N_CTX


def _sc_concat(init, token_prefix, token_suffix):
    info = plsc.get_sparse_core_info()
    NC, NS = info.num_cores, info.num_subcores
    NW = NC * NS
    BASE = N_CLS // NW
    REM = N_CLS - BASE * NW

    mesh = plsc.VectorSubcoreMesh(core_axis_name="c", subcore_axis_name="s")

    @functools.partial(
        pl.kernel,
        mesh=mesh,
        out_type=jax.ShapeDtypeStruct((N_CLS, SEQ, D), jnp.float32),
        scratch_types=[
            pltpu.VMEM_SHARED((BASE + 1, N_CTX, D), jnp.float32),
            pltpu.SemaphoreType.DMA,
            pltpu.SemaphoreType.DMA,
        ],
        compiler_params=pltpu.CompilerParams(use_tc_tiling_on_sc=False),
    )
    def k(ctx_hbm, pre_hbm, suf_hbm, out_hbm, ctx_rep, sem_io, sem_ctx):
        cid = lax.axis_index("c")
        sid = lax.axis_index("s")
        wid = sid * NC + cid
        lo = wid * BASE

        pltpu.sync_copy(ctx_hbm, ctx_rep.at[2 * sid])
        pltpu.sync_copy(ctx_hbm, ctx_rep.at[2 * sid + 1])
        plsc.subcore_barrier()

        pre_cp = pltpu.async_copy(
            pre_hbm.at[pl.ds(lo, BASE)],
            out_hbm.at[pl.ds(lo, BASE), pl.ds(0, 1)],
            sem_io,
        )
        suf_cp = pltpu.async_copy(
            suf_hbm.at[pl.ds(lo, BASE)],
            out_hbm.at[pl.ds(lo, BASE), pl.ds(1 + N_CTX, SUF)],
            sem_io,
        )
        ctx_cp = pltpu.async_copy(
            ctx_rep.at[pl.ds(0, BASE)],
            out_hbm.at[pl.ds(lo, BASE), pl.ds(1, N_CTX)],
            sem_ctx,
        )

        @pl.when(wid < REM)
        def _():
            c = NW * BASE + wid
            pltpu.async_copy(
                pre_hbm.at[c], out_hbm.at[c, pl.ds(0, 1)], sem_io
            ).wait()
            pltpu.async_copy(
                suf_hbm.at[c], out_hbm.at[c, pl.ds(1 + N_CTX, SUF)], sem_io
            ).wait()
            pltpu.async_copy(
                ctx_rep.at[BASE], out_hbm.at[c, pl.ds(1, N_CTX)], sem_ctx
            ).wait()

        pre_cp.wait()
        suf_cp.wait()
        ctx_cp.wait()

    return k(init, token_prefix, token_suffix)


def kernel(init, token_prefix, token_suffix):
    return _sc_concat(init, token_prefix, token_suffix)


# speedup vs baseline: 9.8872x; 9.8872x over previous
import functools

import jax
import jax.numpy as jnp
from jax import lax
from jax.experimental import pallas as pl
from jax.experimental.pallas import tpu as pltpu
from jax.experimental.pallas import tpu_sc as plsc

N_CLS = 1000
N_CTX = 16
D = 768
SEQ = 77
SUF = SEQ - 1 - N_CTX
LANES = 16
NJ = D // LANES


def _sc_concat(init, token_prefix, token_suffix):
    info = plsc.get_sparse_core_info()
    NC, NS = info.num_cores, info.num_subcores
    NW = NC * NS
    BASE = N_CLS // NW
    REM = N_CLS - BASE * NW

    mesh = plsc.VectorSubcoreMesh(core_axis_name="c", subcore_axis_name="s")

    @functools.partial(
        pl.kernel,
        mesh=mesh,
        out_type=jax.ShapeDtypeStruct((N_CLS, SEQ, D), jnp.float32),
        scratch_types=[
            pltpu.VMEM((N_CTX, D), jnp.float32),
            pltpu.VMEM((1, D), jnp.float32),
            pltpu.VMEM((SUF, D), jnp.float32),
            pltpu.VMEM((SEQ, D), jnp.float32),
        ],
    )
    def k(ctx_hbm, pre_hbm, suf_hbm, out_hbm, ctx_v, pre_v, suf_v, out_v):
        cid = lax.axis_index("c")
        sid = lax.axis_index("s")
        wid = sid * NC + cid
        lo = wid * BASE

        pltpu.sync_copy(ctx_hbm, ctx_v)

        def ctx_row(r, carry):
            for j in range(NJ):
                out_v[1 + r, pl.ds(j * LANES, LANES)] = ctx_v[r, pl.ds(j * LANES, LANES)]
            return carry

        lax.fori_loop(0, N_CTX, ctx_row, 0)

        def body(c, carry):
            pltpu.sync_copy(pre_hbm.at[c], pre_v)
            pltpu.sync_copy(suf_hbm.at[c], suf_v)
            for j in range(NJ):
                out_v[0, pl.ds(j * LANES, LANES)] = pre_v[0, pl.ds(j * LANES, LANES)]

            def suf_row(r, carry2):
                for j in range(NJ):
                    out_v[1 + N_CTX + r, pl.ds(j * LANES, LANES)] = suf_v[r, pl.ds(j * LANES, LANES)]
                return carry2

            lax.fori_loop(0, SUF, suf_row, 0)
            pltpu.sync_copy(out_v, out_hbm.at[c])
            return carry

        lax.fori_loop(lo, lo + BASE, body, 0)

        @pl.when(wid < REM)
        def _():
            lax.fori_loop(NW * BASE + wid, NW * BASE + wid + 1, body, 0)

    return k(init, token_prefix, token_suffix)


def kernel(init, token_prefix, token_suffix):
    return _sc_concat(init, token_prefix, token_suffix)


# speedup vs baseline: 15.0819x; 1.5254x over previous
import functools

import jax
import jax.numpy as jnp
from jax import lax
from jax.experimental import pallas as pl
from jax.experimental.pallas import tpu as pltpu
from jax.experimental.pallas import tpu_sc as plsc

N_CLS = 1000
N_CTX = 16
D = 768
SEQ = 77
SUF = SEQ - 1 - N_CTX
LANES = 16
NJ = D // LANES

SUF_A = 32
SUF_B = SUF - SUF_A
OUT_P1 = 48
OUT_P2 = SEQ - OUT_P1


def _sc_concat(init, token_prefix, token_suffix):
    info = plsc.get_sparse_core_info()
    NC, NS = info.num_cores, info.num_subcores
    NW = NC * NS
    BASE = N_CLS // NW
    REM = N_CLS - BASE * NW

    mesh = plsc.VectorSubcoreMesh(core_axis_name="c", subcore_axis_name="s")

    @functools.partial(
        pl.kernel,
        mesh=mesh,
        out_type=jax.ShapeDtypeStruct((N_CLS, SEQ, D), jnp.float32),
        scratch_types=[
            pltpu.VMEM((N_CTX, D), jnp.float32),
            pltpu.VMEM((1, D), jnp.float32),
            pltpu.VMEM((SUF_A, D), jnp.float32),
            pltpu.VMEM((SUF_B, D), jnp.float32),
            pltpu.VMEM((SEQ, D), jnp.float32),
            pltpu.SemaphoreType.DMA,
            pltpu.SemaphoreType.DMA,
            pltpu.SemaphoreType.DMA,
            pltpu.SemaphoreType.DMA,
            pltpu.SemaphoreType.DMA,
        ],
    )
    def k(ctx_hbm, pre_hbm, suf_hbm, out_hbm,
          ctx_v, pre_v, suf_a, suf_b, out_v, s_a, s_b, s_p, s_o1, s_o2):
        cid = lax.axis_index("c")
        sid = lax.axis_index("s")
        wid = sid * NC + cid
        lo = wid * BASE + jnp.minimum(wid, REM)
        hi = lo + BASE + jnp.where(wid < REM, 1, 0)

        pltpu.sync_copy(ctx_hbm, ctx_v)
        for r in range(N_CTX):
            for j in range(NJ):
                out_v[1 + r, pl.ds(j * LANES, LANES)] = ctx_v[r, pl.ds(j * LANES, LANES)]

        pltpu.async_copy(suf_hbm.at[lo, pl.ds(0, SUF_A)], suf_a, s_a)
        pltpu.async_copy(pre_hbm.at[lo], pre_v, s_p)

        def body(c, carry):
            nxt = jnp.minimum(c + 1, N_CLS - 1)

            pltpu.make_async_copy(suf_hbm.at[c, pl.ds(0, SUF_A)], suf_a, s_a).wait()
            pltpu.make_async_copy(pre_hbm.at[c], pre_v, s_p).wait()
            pltpu.async_copy(suf_hbm.at[c, pl.ds(SUF_A, SUF_B)], suf_b, s_b)

            @pl.when(c > lo)
            def _():
                pltpu.make_async_copy(
                    out_v.at[pl.ds(0, OUT_P1)],
                    out_hbm.at[c, pl.ds(0, OUT_P1)], s_o1).wait()

            for j in range(NJ):
                out_v[0, pl.ds(j * LANES, LANES)] = pre_v[0, pl.ds(j * LANES, LANES)]
            for r in range(SUF_A):
                for j in range(NJ):
                    out_v[1 + N_CTX + r, pl.ds(j * LANES, LANES)] = \
                        suf_a[r, pl.ds(j * LANES, LANES)]

            pltpu.async_copy(out_v.at[pl.ds(0, OUT_P1)],
                             out_hbm.at[c, pl.ds(0, OUT_P1)], s_o1)
            pltpu.async_copy(pre_hbm.at[nxt], pre_v, s_p)

            pltpu.make_async_copy(suf_hbm.at[c, pl.ds(SUF_A, SUF_B)], suf_b, s_b).wait()

            @pl.when(c > lo)
            def _():
                pltpu.make_async_copy(
                    out_v.at[pl.ds(OUT_P1, OUT_P2)],
                    out_hbm.at[c, pl.ds(OUT_P1, OUT_P2)], s_o2).wait()

            for r in range(SUF_B):
                for j in range(NJ):
                    out_v[1 + N_CTX + SUF_A + r, pl.ds(j * LANES, LANES)] = \
                        suf_b[r, pl.ds(j * LANES, LANES)]

            pltpu.async_copy(out_v.at[pl.ds(OUT_P1, OUT_P2)],
                             out_hbm.at[c, pl.ds(OUT_P1, OUT_P2)], s_o2)
            pltpu.async_copy(suf_hbm.at[nxt, pl.ds(0, SUF_A)], suf_a, s_a)
            return carry

        lax.fori_loop(lo, hi, body, 0)

        pltpu.make_async_copy(out_v.at[pl.ds(0, OUT_P1)],
                              out_hbm.at[0, pl.ds(0, OUT_P1)], s_o1).wait()
        pltpu.make_async_copy(out_v.at[pl.ds(OUT_P1, OUT_P2)],
                              out_hbm.at[0, pl.ds(OUT_P1, OUT_P2)], s_o2).wait()
        pltpu.make_async_copy(suf_hbm.at[0, pl.ds(0, SUF_A)], suf_a, s_a).wait()
        pltpu.make_async_copy(pre_hbm.at[0], pre_v, s_p).wait()

    return k(init, token_prefix, token_suffix)


def kernel(init, token_prefix, token_suffix):
    return _sc_concat(init, token_prefix, token_suffix)


# speedup vs baseline: 15.5698x; 1.0324x over previous
import functools

import jax
import jax.numpy as jnp
from jax import lax
from jax.experimental import pallas as pl
from jax.experimental.pallas import tpu as pltpu
from jax.experimental.pallas import tpu_sc as plsc

N_CLS = 1000
N_CTX = 16
D = 768
SEQ = 77
SUF = SEQ - 1 - N_CTX
LANES = 16
NJ = D // LANES

SUF_A = 32
SUF_B = SUF - SUF_A
P1 = 48
P2R = SEQ - P1
BIGW = 20


def _sc_concat(init, token_prefix, token_suffix):
    info = plsc.get_sparse_core_info()
    NC, NS = info.num_cores, info.num_subcores
    NW = NC * NS

    mesh = plsc.VectorSubcoreMesh(core_axis_name="c", subcore_axis_name="s")

    @functools.partial(
        pl.kernel,
        mesh=mesh,
        out_type=jax.ShapeDtypeStruct((N_CLS, SEQ, D), jnp.float32),
        scratch_types=[
            pltpu.VMEM((P1, D), jnp.float32),
            pltpu.VMEM((P1, D), jnp.float32),
            pltpu.VMEM((SUF_B, D), jnp.float32),
            pltpu.VMEM((P2R, D), jnp.float32),
            pltpu.VMEM((1, D), jnp.float32),
            pltpu.VMEM((1, D), jnp.float32),
            pltpu.VMEM((1, D), jnp.float32),
            pltpu.VMEM((1, D), jnp.float32),
            pltpu.SemaphoreType.DMA,
            pltpu.SemaphoreType.DMA,
            pltpu.SemaphoreType.DMA,
            pltpu.SemaphoreType.DMA,
            pltpu.SemaphoreType.DMA,
            pltpu.SemaphoreType.DMA,
            pltpu.SemaphoreType.DMA,
            pltpu.SemaphoreType.DMA,
        ],
    )
    def k(ctx_hbm, pre_hbm, suf_hbm, out_hbm,
          a0, a1, sb, p2, ctx15, pre_a, pre_b, tbuf,
          s_ia0, s_ia1, s_isb, s_pa, s_pb, s_sa0, s_sa1, s_sp2):
        cid = lax.axis_index("c")
        sid = lax.axis_index("s")
        wid = sid * NC + cid
        lo = 30 * wid + 2 * jnp.minimum(wid, BIGW)
        npairs = 15 + jnp.where(wid < BIGW, 1, 0)

        def clamp(c):
            return jnp.minimum(c, N_CLS - 1)

        def vrow(dst, dr, src, sr):
            for j in range(NJ):
                dst[dr, pl.ds(j * LANES, LANES)] = src[sr, pl.ds(j * LANES, LANES)]

        def in_a(c, a, sem):
            return pltpu.make_async_copy(
                suf_hbm.at[c, pl.ds(0, SUF_A)], a.at[pl.ds(N_CTX, SUF_A)], sem)

        def in_sb(c, sem):
            return pltpu.make_async_copy(
                suf_hbm.at[c, pl.ds(SUF_A, SUF_B)], sb, sem)

        def in_pre(c, buf, sem):
            return pltpu.make_async_copy(pre_hbm.at[c], buf, sem)

        def st_a(c, a, sem):
            return pltpu.make_async_copy(a, out_hbm.at[c, pl.ds(0, P1)], sem)

        def st_p2(c, sem):
            return pltpu.make_async_copy(p2, out_hbm.at[c, pl.ds(P1, P2R)], sem)

        def shift_a(a):
            vrow(tbuf, 0, a, P1 - 1)
            for r in range(SUF_A - 2, -1, -1):
                vrow(a, 1 + N_CTX + r, a, N_CTX + r)
            vrow(a, N_CTX, ctx15, 0)

        pltpu.sync_copy(ctx_hbm, a0.at[pl.ds(0, N_CTX)])
        vrow(ctx15, 0, a0, N_CTX - 1)
        for r in range(N_CTX - 2, -1, -1):
            vrow(a0, 1 + r, a0, r)
        for r in range(1, N_CTX):
            vrow(a1, r, a0, r)

        in_a(lo, a0, s_ia0).start()
        in_a(lo + 1, a1, s_ia1).start()
        in_sb(lo, s_isb).start()
        in_pre(lo, pre_a, s_pa).start()
        in_pre(lo + 1, pre_b, s_pb).start()

        def pair(p, carry):
            c0 = lo + 2 * p
            c1 = c0 + 1

            @pl.when(p > 0)
            def _():
                st_a(c0, a1, s_sa1).wait()
                in_a(c1, a1, s_ia1).start()

            in_a(c0, a0, s_ia0).wait()
            shift_a(a0)
            in_pre(c0, pre_a, s_pa).wait()
            vrow(a0, 0, pre_a, 0)
            st_a(c0, a0, s_sa0).start()
            in_pre(clamp(c0 + 2), pre_a, s_pa).start()

            @pl.when(p > 0)
            def _():
                st_p2(c0, s_sp2).wait()
            vrow(p2, 0, tbuf, 0)
            in_sb(c0, s_isb).wait()
            for q in range(SUF_B):
                vrow(p2, 1 + q, sb, q)
            st_p2(c0, s_sp2).start()
            in_sb(c1, s_isb).start()

            in_a(c1, a1, s_ia1).wait()
            shift_a(a1)
            in_pre(c1, pre_b, s_pb).wait()
            vrow(a1, 0, pre_b, 0)
            st_a(c1, a1, s_sa1).start()
            in_pre(clamp(c1 + 2), pre_b, s_pb).start()

            st_a(c0, a0, s_sa0).wait()
            in_a(clamp(c0 + 2), a0, s_ia0).start()

            st_p2(c0, s_sp2).wait()
            vrow(p2, 0, tbuf, 0)
            in_sb(c1, s_isb).wait()
            for q in range(SUF_B):
                vrow(p2, 1 + q, sb, q)
            st_p2(c1, s_sp2).start()
            in_sb(clamp(c1 + 1), s_isb).start()
            return carry

        lax.fori_loop(0, npairs, pair, 0)

        st_a(0, a1, s_sa1).wait()
        st_p2(0, s_sp2).wait()
        in_a(0, a0, s_ia0).wait()
        in_sb(0, s_isb).wait()
        in_pre(0, pre_a, s_pa).wait()
        in_pre(0, pre_b, s_pb).wait()

    return k(init, token_prefix, token_suffix)


def kernel(init, token_prefix, token_suffix):
    return _sc_concat(init, token_prefix, token_suffix)
